# baseline (speedup 1.0000x reference)
import jax
import jax.numpy as jnp
import numpy as np
from jax.experimental import pallas as pl
from jax.experimental.pallas import tpu as pltpu

_B = 4096
_D = 1024
_L = 64
_K = 8
_C = 1024
_TB = 128
_GRID = _B // _TB
_KTB = _K * _TB
_A = _L + _TB
_CC = 64


def _step(encT_ref, dinT_ref, lab_ref, wencT_ref, wdzT_ref, woutT_ref,
          abot_ref, loss_ref, ent_ref, a_scr, p_scr, e_scr):
    i = pl.program_id(0)

    dslot = (i - 1) % 2
    rhsT = jnp.concatenate([wdzT_ref[...], dinT_ref[...]], axis=1)
    h_preT = jnp.dot(rhsT, a_scr[dslot],
                     preferred_element_type=jnp.float32)
    hT = jnp.maximum(h_preT, 0.0).astype(jnp.float8_e4m3fn)

    lab_t = jnp.concatenate([lab_ref[0]] * _K, axis=1)
    se = jnp.zeros((1, _KTB), jnp.float32)
    dlab = jnp.zeros((1, _KTB), jnp.float32)
    for t in range(_C // _CC):
        dt = jnp.dot(woutT_ref[t * _CC:(t + 1) * _CC, :], hT,
                     preferred_element_type=jnp.float32)
        se = se + jnp.sum(jnp.exp2(dt), axis=0, keepdims=True)
        iota_cc = (jax.lax.broadcasted_iota(jnp.int32, (_CC, _KTB), 0)
                   + t * _CC)
        dlab = dlab + jnp.sum(
            jnp.where(iota_cc == lab_t, dt, 0.0),
            axis=0, keepdims=True)
    loss_row = jnp.log(se) - dlab * jnp.float32(0.6931471805599453)

    partial = jnp.sum(p_scr[dslot] * loss_row)
    loss_ref[...] = jnp.broadcast_to(partial, (1, 1, 128))
    ent_ref[...] = e_scr[dslot][None]

    logitsT = jnp.dot(wencT_ref[...], encT_ref[...],
                      preferred_element_type=jnp.float32)

    iota_s = jax.lax.broadcasted_iota(jnp.int32, (_L, _TB), 0)
    v = logitsT
    zs, idxs = [], []
    for _ in range(_K):
        m = jnp.max(v, axis=0, keepdims=True)
        i_k = jnp.min(jnp.where(v == m, iota_s, _L), axis=0, keepdims=True)
        v = jnp.where(iota_s == i_k, -1e30, v)
        zs.append(m)
        idxs.append(i_k)

    cs = jnp.zeros((1, _TB), jnp.float32)
    ksup = jnp.zeros((1, _TB), jnp.float32)
    cssel = jnp.zeros((1, _TB), jnp.float32)
    for k in range(_K):
        cs = cs + zs[k]
        sup = ((1.0 + (k + 1) * zs[k]) > cs).astype(jnp.float32)
        ksup = ksup + sup
        cssel = cssel + sup * zs[k]
    tau = (cssel - 1.0) / ksup
    ps = [jnp.maximum(zs[k] - tau, 0.0) for k in range(_K)]
    ent = 0.0
    for k in range(_K):
        ent = ent + jnp.sum(-ps[k] * jnp.log(ps[k] + 1e-10))

    i_row = jnp.concatenate(idxs, axis=1)
    at_top = (jax.lax.broadcasted_iota(jnp.int32, (_L, _KTB), 0)
              == i_row).astype(jnp.bfloat16)
    rslot = i % 2
    a_scr[rslot] = jnp.concatenate([at_top, abot_ref[...]], axis=0)
    p_scr[rslot] = jnp.concatenate(ps, axis=1)
    e_scr[rslot] = jnp.broadcast_to(ent, (1, 128))


def kernel(encoder_input, decoder_input, labels, W_enc, W_dec_z, W_dec_out):
    encT16 = encoder_input.T.astype(jnp.bfloat16)
    dinT16 = decoder_input.T.astype(jnp.bfloat16)
    wencT16 = W_enc.T.astype(jnp.bfloat16)
    wdzT16 = W_dec_z.T.astype(jnp.bfloat16)
    woutT16 = (W_dec_out.T * np.log2(np.e)).astype(jnp.float8_e4m3fn)
    abot = (jnp.arange(_TB, dtype=jnp.int32)[:, None]
            == (jnp.arange(_KTB, dtype=jnp.int32)[None, :] & (_TB - 1))
            ).astype(jnp.bfloat16)
    lab3 = labels.astype(jnp.int32).reshape(_GRID, 1, _TB)

    loss_p, ent_p = pl.pallas_call(
        _step,
        grid=(_GRID + 1,),
        in_specs=[
            pl.BlockSpec((_D, _TB), lambda i: (0, jnp.minimum(i, _GRID - 1))),
            pl.BlockSpec((_D, _TB), lambda i: (0, jnp.maximum(i - 1, 0))),
            pl.BlockSpec((1, 1, _TB), lambda i: (jnp.maximum(i - 1, 0), 0, 0)),
            pl.BlockSpec((_L, _D), lambda i: (0, 0)),
            pl.BlockSpec((_D, _L), lambda i: (0, 0)),
            pl.BlockSpec((_C, _D), lambda i: (0, 0)),
            pl.BlockSpec((_TB, _KTB), lambda i: (0, 0)),
        ],
        out_specs=[
            pl.BlockSpec((1, 1, 128),
                         lambda i: (jnp.maximum(i - 1, 0), 0, 0)),
            pl.BlockSpec((1, 1, 128),
                         lambda i: (jnp.maximum(i - 1, 0), 0, 0)),
        ],
        out_shape=[
            jax.ShapeDtypeStruct((_GRID, 1, 128), jnp.float32),
            jax.ShapeDtypeStruct((_GRID, 1, 128), jnp.float32),
        ],
        scratch_shapes=[
            pltpu.VMEM((2, _A, _KTB), jnp.bfloat16),
            pltpu.VMEM((2, 1, _KTB), jnp.float32),
            pltpu.VMEM((2, 1, 128), jnp.float32),
        ],
        compiler_params=pltpu.CompilerParams(
            dimension_semantics=("arbitrary",)),
    )(encT16, dinT16, lab3, wencT16, wdzT16, woutT16, abot)

    total = jnp.sum(loss_p[:, 0, 0]) - 0.01 * jnp.sum(ent_p[:, 0, 0])
    return (total / _B).reshape(())


# speedup vs baseline: 1.2749x; 1.2749x over previous
import jax
import jax.numpy as jnp
import numpy as np
from jax.experimental import pallas as pl
from jax.experimental.pallas import tpu as pltpu

_B = 4096
_D = 1024
_L = 64
_K = 8
_C = 1024
_TB = 128
_GRID = _B // _TB
_KTB = _K * _TB
_A = _L + _TB
_CC = 128


def _step(encT_ref, dinT_ref, lab_ref, wencT_ref, wdzT_ref, woutT_ref,
          abot_ref, loss_ref, ent_ref, a_scr, p_scr, e_scr):
    i = pl.program_id(0)

    dslot = (i - 1) % 2
    rhsT = jnp.concatenate([wdzT_ref[...], dinT_ref[...]], axis=1)
    h_preT = jnp.dot(rhsT, a_scr[dslot],
                     preferred_element_type=jnp.float32)
    hT = jnp.maximum(h_preT, 0.0).astype(jnp.float8_e4m3fn)

    lab_t = jnp.concatenate([lab_ref[0]] * _K, axis=1)
    se = jnp.zeros((1, _KTB), jnp.float32)
    dlab = jnp.zeros((1, _KTB), jnp.float32)
    for t in range(_C // _CC):
        dt = jnp.dot(woutT_ref[t * _CC:(t + 1) * _CC, :], hT,
                     preferred_element_type=jnp.float32)
        se = se + jnp.sum(jnp.exp2(dt), axis=0, keepdims=True)
        iota_cc = (jax.lax.broadcasted_iota(jnp.int32, (_CC, _KTB), 0)
                   + t * _CC)
        dlab = dlab + jnp.sum(
            jnp.where(iota_cc == lab_t, dt, 0.0),
            axis=0, keepdims=True)
    loss_row = jnp.log(se) - dlab * jnp.float32(0.6931471805599453)

    partial = jnp.sum(p_scr[dslot] * loss_row)
    loss_ref[...] = jnp.broadcast_to(partial, (1, 1, 128))
    ent_ref[...] = e_scr[dslot][None]

    logitsT = jnp.dot(wencT_ref[...], encT_ref[...],
                      preferred_element_type=jnp.float32)

    iota_s = jax.lax.broadcasted_iota(jnp.int32, (_L, _TB), 0)
    v = logitsT
    zs, idxs = [], []
    for _ in range(_K):
        m = jnp.max(v, axis=0, keepdims=True)
        i_k = jnp.min(jnp.where(v == m, iota_s, _L), axis=0, keepdims=True)
        v = jnp.where(iota_s == i_k, -1e30, v)
        zs.append(m)
        idxs.append(i_k)

    cs = jnp.zeros((1, _TB), jnp.float32)
    ksup = jnp.zeros((1, _TB), jnp.float32)
    cssel = jnp.zeros((1, _TB), jnp.float32)
    for k in range(_K):
        cs = cs + zs[k]
        sup = ((1.0 + (k + 1) * zs[k]) > cs).astype(jnp.float32)
        ksup = ksup + sup
        cssel = cssel + sup * zs[k]
    tau = (cssel - 1.0) / ksup
    ps = [jnp.maximum(zs[k] - tau, 0.0) for k in range(_K)]
    ent = 0.0
    for k in range(_K):
        ent = ent + jnp.sum(-ps[k] * jnp.log(ps[k] + 1e-10))

    i_row = jnp.concatenate(idxs, axis=1)
    at_top = (jax.lax.broadcasted_iota(jnp.int32, (_L, _KTB), 0)
              == i_row).astype(jnp.bfloat16)
    rslot = i % 2
    a_scr[rslot] = jnp.concatenate([at_top, abot_ref[...]], axis=0)
    p_scr[rslot] = jnp.concatenate(ps, axis=1)
    e_scr[rslot] = jnp.broadcast_to(ent, (1, 128))


def kernel(encoder_input, decoder_input, labels, W_enc, W_dec_z, W_dec_out):
    encT16 = encoder_input.T.astype(jnp.bfloat16)
    dinT16 = decoder_input.T.astype(jnp.bfloat16)
    wencT16 = W_enc.T.astype(jnp.bfloat16)
    wdzT16 = W_dec_z.T.astype(jnp.bfloat16)
    woutT16 = (W_dec_out.T * np.log2(np.e)).astype(jnp.float8_e4m3fn)
    abot = (jnp.arange(_TB, dtype=jnp.int32)[:, None]
            == (jnp.arange(_KTB, dtype=jnp.int32)[None, :] & (_TB - 1))
            ).astype(jnp.bfloat16)
    lab3 = labels.astype(jnp.int32).reshape(_GRID, 1, _TB)

    loss_p, ent_p = pl.pallas_call(
        _step,
        grid=(_GRID + 1,),
        in_specs=[
            pl.BlockSpec((_D, _TB), lambda i: (0, jnp.minimum(i, _GRID - 1))),
            pl.BlockSpec((_D, _TB), lambda i: (0, jnp.maximum(i - 1, 0))),
            pl.BlockSpec((1, 1, _TB), lambda i: (jnp.maximum(i - 1, 0), 0, 0)),
            pl.BlockSpec((_L, _D), lambda i: (0, 0)),
            pl.BlockSpec((_D, _L), lambda i: (0, 0)),
            pl.BlockSpec((_C, _D), lambda i: (0, 0)),
            pl.BlockSpec((_TB, _KTB), lambda i: (0, 0)),
        ],
        out_specs=[
            pl.BlockSpec((1, 1, 128),
                         lambda i: (jnp.maximum(i - 1, 0), 0, 0)),
            pl.BlockSpec((1, 1, 128),
                         lambda i: (jnp.maximum(i - 1, 0), 0, 0)),
        ],
        out_shape=[
            jax.ShapeDtypeStruct((_GRID, 1, 128), jnp.float32),
            jax.ShapeDtypeStruct((_GRID, 1, 128), jnp.float32),
        ],
        scratch_shapes=[
            pltpu.VMEM((2, _A, _KTB), jnp.bfloat16),
            pltpu.VMEM((2, 1, _KTB), jnp.float32),
            pltpu.VMEM((2, 1, 128), jnp.float32),
        ],
        compiler_params=pltpu.CompilerParams(
            dimension_semantics=("arbitrary",)),
    )(encT16, dinT16, lab3, wencT16, wdzT16, woutT16, abot)

    total = jnp.sum(loss_p[:, 0, 0]) - 0.01 * jnp.sum(ent_p[:, 0, 0])
    return (total / _B).reshape(())
